# baseline (speedup 1.0000x reference)
import jax
import jax.numpy as jnp
from jax.experimental import pallas as pl


def _bias_body(x_ref, b_ref, o_ref):
    o_ref[...] = x_ref[...] + b_ref[...]


def _bias_add(x, b):
    n, d = x.shape
    blk = 1000
    return pl.pallas_call(
        _bias_body,
        grid=(n // blk,),
        in_specs=[
            pl.BlockSpec((blk, d), lambda i: (i, 0)),
            pl.BlockSpec((1, d), lambda i: (0, 0)),
        ],
        out_specs=pl.BlockSpec((blk, d), lambda i: (i, 0)),
        out_shape=jax.ShapeDtypeStruct((n, d), x.dtype),
    )(x, b.reshape(1, d))


def _gat(x_in, src, dst, W, att_src, att_dst, bias, heads, out_ch, concat):
    n = x_in.shape[0]
    h = (x_in @ W).reshape(n, heads, out_ch)
    a_src = jnp.sum(h * att_src, axis=-1)
    a_dst = jnp.sum(h * att_dst, axis=-1)
    e = a_src[src] + a_dst[dst]
    e = jax.nn.leaky_relu(e, negative_slope=0.2)
    m = jax.ops.segment_max(e, dst, num_segments=n)
    ex = jnp.exp(e - m[dst])
    den = jax.ops.segment_sum(ex, dst, num_segments=n)
    alpha = ex / (den[dst] + 1e-16)
    msg = h[src] * alpha[:, :, None]
    out = jax.ops.segment_sum(msg, dst, num_segments=n)
    if concat:
        out = out.reshape(n, heads * out_ch)
    else:
        out = out.mean(axis=1)
    return _bias_add(out, bias)


def kernel(x, edge_index, W1, as1, ad1, b1, W2, as2, ad2, b2, Wm, asm, adm, bm, Wl, asl, adl, bl):
    n = x.shape[0]
    loops = jnp.arange(n, dtype=edge_index.dtype)
    src = jnp.concatenate([edge_index[0], loops])
    dst = jnp.concatenate([edge_index[1], loops])
    h = _gat(x, src, dst, W1, as1, ad1, b1, heads=2, out_ch=512, concat=True)
    h = jax.nn.relu(h)
    h = _gat(h, src, dst, W2, as2, ad2, b2, heads=2, out_ch=512, concat=True)
    h = jax.nn.relu(h)
    mu = _gat(h, src, dst, Wm, asm, adm, bm, heads=1, out_ch=256, concat=False)
    logstd = _gat(h, src, dst, Wl, asl, adl, bl, heads=1, out_ch=256, concat=False)
    return (mu, logstd)


# speedup vs baseline: 18.8187x; 18.8187x over previous
import functools

import jax
import jax.numpy as jnp
from jax import lax
from jax.experimental import pallas as pl
from jax.experimental.pallas import tpu as pltpu
from jax.experimental.pallas import tpu_sc as plsc

N = 10000
NP = 10240
NC = 2
NS = 16
LANES = 16
G = 96
BLK = 1000
STRIPE = NP // NS


def _mm_body(cph, x_ref, w_ref, as_ref, ad_ref, h_ref, s_ref, d_ref):
    c = pl.program_id(1)
    hv = jnp.dot(x_ref[...], w_ref[...], preferred_element_type=jnp.float32)
    h_ref[0] = hv
    head = c // cph
    nh = s_ref.shape[1]
    onehot = (lax.broadcasted_iota(jnp.int32, (1, nh), 1) == head).astype(jnp.float32)
    sc = jnp.sum(hv * as_ref[pl.ds(c, 1), :], axis=1, keepdims=True)
    dc = jnp.sum(hv * ad_ref[pl.ds(c, 1), :], axis=1, keepdims=True)

    @pl.when(c == 0)
    def _():
        s_ref[...] = jnp.zeros_like(s_ref)
        d_ref[...] = jnp.zeros_like(d_ref)

    s_ref[...] += sc * onehot
    d_ref[...] += dc * onehot


def _matmul_att(x, w, asv, adv, ch, cph):
    n, k = x.shape
    nh = ch // cph
    nb = n // BLK
    return pl.pallas_call(
        functools.partial(_mm_body, cph),
        grid=(nb, ch),
        in_specs=[
            pl.BlockSpec((BLK, k), lambda i, c: (i, 0)),
            pl.BlockSpec((k, 128), lambda i, c: (0, c)),
            pl.BlockSpec((ch, 128), lambda i, c: (0, 0)),
            pl.BlockSpec((ch, 128), lambda i, c: (0, 0)),
        ],
        out_specs=[
            pl.BlockSpec((1, BLK, 128), lambda i, c: (c, i, 0)),
            pl.BlockSpec((BLK, nh), lambda i, c: (i, 0)),
            pl.BlockSpec((BLK, nh), lambda i, c: (i, 0)),
        ],
        out_shape=[
            jax.ShapeDtypeStruct((ch, n, 128), jnp.float32),
            jax.ShapeDtypeStruct((n, nh), jnp.float32),
            jax.ShapeDtypeStruct((n, nh), jnp.float32),
        ],
    )(x, w, asv, adv)


def _fin_body(do_relu, a_ref, b_ref, o_ref):
    c = pl.program_id(1)
    v = a_ref[0] + b_ref[pl.ds(c, 1), :]
    if do_relu:
        v = jnp.maximum(v, 0.0)
    o_ref[...] = v


def _finish(agg, b2d, do_relu):
    ch, n, _ = agg.shape
    nb = n // BLK
    return pl.pallas_call(
        functools.partial(_fin_body, do_relu),
        grid=(nb, ch),
        in_specs=[
            pl.BlockSpec((1, BLK, 128), lambda i, c: (c, i, 0)),
            pl.BlockSpec((ch, 128), lambda i, c: (0, 0)),
        ],
        out_specs=pl.BlockSpec((BLK, 128), lambda i, c: (i, c)),
        out_shape=jax.ShapeDtypeStruct((n, ch * 128), jnp.float32),
    )(agg, b2d)


def _gat_sc_body(ch, cph, ec, e_tot,
                 h_hbm, as_hbm, ad_hbm, sd_hbm, z_hbm, out_hbm,
                 den2, exv, sdb0, sdb1, gidx0, gidx1, didx, kbuf, cbuf,
                 acc_sh, sem_s0, sem_s1, sem_g0, sem_g1):
    core = lax.axis_index("c")
    s = lax.axis_index("s")
    head = core
    ngroups = ec // G
    npairs = ngroups // 2
    ebase = s * ec
    sdbase = s * ngroups * 2 * G
    io16 = lax.broadcasted_iota(jnp.int32, (LANES,), 0)
    NSB = G // LANES

    def _sd_start(g, sdb, sem):
        pltpu.async_copy(sd_hbm.at[pl.ds(sdbase + g * 2 * G, 2 * G)], sdb, sem)

    def _sd_wait(sdb, sem):
        pltpu.make_async_copy(sd_hbm.at[pl.ds(0, 2 * G)], sdb, sem).wait()

    def _allmax(v):
        for k in (1, 2, 4, 8):
            cbuf[...] = v
            v = jnp.maximum(v, plsc.load_gather(cbuf, [io16 ^ k]))
        return v

    def _cumsum16(v):
        for k in (1, 2, 4, 8):
            cbuf[...] = v
            t = plsc.load_gather(cbuf, [jnp.maximum(io16 - k, 0)])
            v = v + jnp.where(io16 >= k, t, 0.0)
        return v

    def _cummax16i(v):
        for k in (1, 2, 4, 8):
            kbuf[...] = v
            t = plsc.load_gather(kbuf, [jnp.maximum(io16 - k, 0)])
            v = jnp.maximum(v, jnp.where(io16 >= k, t, 0))
        return v

    def _z_den(i, _):
        den2[i // 8, pl.ds((i % 8) * LANES, LANES)] = jnp.zeros(
            (LANES,), jnp.float32)
        return 0
    lax.fori_loop(0, NP // LANES, _z_den, 0)

    def _phase1(asrc_t, adst_t):
        pltpu.sync_copy(as_hbm.at[pl.ds(head * NP, NP)], asrc_t)
        pltpu.sync_copy(ad_hbm.at[pl.ds(head * NP, NP)], adst_t)

        def _mx(i, carry):
            ms, md = carry
            ms = jnp.maximum(ms, asrc_t[pl.ds(i * LANES, LANES)])
            md = jnp.maximum(md, adst_t[pl.ds(i * LANES, LANES)])
            return ms, md
        neg = jnp.full((LANES,), -3.0e38, jnp.float32)
        ms, md = lax.fori_loop(0, N // LANES, _mx, (neg, neg))
        m_bound = jnp.maximum(_allmax(ms)[0] + _allmax(md)[0], 0.0)

        def _proc1(g, sdb):
            def _inner(sb, _):
                ii = pl.ds(g * G + sb * LANES, LANES)
                sv = sdb[pl.ds(sb * LANES, LANES)]
                dv = sdb[pl.ds(G + sb * LANES, LANES)]
                av = plsc.load_gather(asrc_t, [sv])
                bv = plsc.load_gather(adst_t, [dv])
                e = av + bv
                e = jnp.where(e >= 0.0, e, 0.2 * e)
                exu = jnp.exp(e - m_bound)
                eid = ebase + g * G + sb * LANES + io16
                exu = jnp.where(eid < e_tot, exu, 0.0)
                exv[ii] = exu
                kk, vv = plsc.sort_key_val(dv, exu)
                cs = _cumsum16(vv)
                kbuf[...] = kk
                knext = plsc.load_gather(
                    kbuf, [jnp.minimum(io16 + 1, LANES - 1)])
                kprev = plsc.load_gather(kbuf, [jnp.maximum(io16 - 1, 0)])
                is_last = (kk != knext) | (io16 == LANES - 1)
                is_first = (kk != kprev) | (io16 == 0)
                start = _cummax16i(jnp.where(is_first, io16, 0))
                cbuf[...] = cs
                base = plsc.load_gather(cbuf, [jnp.maximum(start - 1, 0)])
                base = jnp.where(start > 0, base, 0.0)
                plsc.addupdate_scatter(
                    den2, [lax.shift_right_logical(kk, 7), kk & 127],
                    cs - base, mask=is_last)
                return 0
            lax.fori_loop(0, NSB, _inner, 0)

        _sd_start(0, sdb0, sem_s0)
        _sd_start(1, sdb1, sem_s1)

        def _pair(k, _):
            g = 2 * k
            _sd_wait(sdb0, sem_s0)
            _proc1(g, sdb0)

            @pl.when(k < npairs - 1)
            def _():
                _sd_start(g + 2, sdb0, sem_s0)
            _sd_wait(sdb1, sem_s1)
            _proc1(g + 1, sdb1)

            @pl.when(k < npairs - 1)
            def _():
                _sd_start(g + 3, sdb1, sem_s1)
            return 0
        lax.fori_loop(0, npairs, _pair, 0)

    pl.run_scoped(_phase1,
                  pltpu.VMEM((NP,), jnp.float32),
                  pltpu.VMEM((NP,), jnp.float32))

    def _phase2(rows0, rows1):
        DB = NP // 128
        DR = STRIPE // 128
        pltpu.sync_copy(den2, acc_sh.at[pl.ds(s * DB, DB)])
        plsc.subcore_barrier()
        pltpu.sync_copy(acc_sh.at[pl.ds(s * DR, DR)], rows0.at[pl.ds(0, DR)])
        for t in range(1, NS):
            pltpu.sync_copy(acc_sh.at[pl.ds(t * DB + s * DR, DR)],
                            rows0.at[pl.ds(8, DR)])

            def _acc(i, _):
                r, j = i // 8, i % 8
                jj = pl.ds(j * LANES, LANES)
                rows0[r, jj] += rows0[8 + r, jj]
                return 0
            lax.fori_loop(0, DR * 8, _acc, 0)
        plsc.subcore_barrier()
        pltpu.sync_copy(rows0.at[pl.ds(0, DR)], acc_sh.at[pl.ds(s * DR, DR)])
        plsc.subcore_barrier()
        pltpu.sync_copy(acc_sh.at[pl.ds(0, DB)], den2)

        def _procc(g, sdb):
            def _inner(sb, _):
                ii = pl.ds(g * G + sb * LANES, LANES)
                dv = sdb[pl.ds(G + sb * LANES, LANES)]
                den_v = plsc.load_gather(
                    den2, [lax.shift_right_logical(dv, 7), dv & 127])
                exv[ii] = exv[ii] / (den_v + 1e-16)
                return 0
            lax.fori_loop(0, NSB, _inner, 0)

        _sd_start(0, sdb0, sem_s0)
        _sd_start(1, sdb1, sem_s1)

        def _pairc(k, _):
            g = 2 * k
            _sd_wait(sdb0, sem_s0)
            _procc(g, sdb0)

            @pl.when(k < npairs - 1)
            def _():
                _sd_start(g + 2, sdb0, sem_s0)
            _sd_wait(sdb1, sem_s1)
            _procc(g + 1, sdb1)

            @pl.when(k < npairs - 1)
            def _():
                _sd_start(g + 3, sdb1, sem_s1)
            return 0
        lax.fori_loop(0, npairs, _pairc, 0)

        def _mk_gidx(sdb, gidx, c):
            def _idx(j, _):
                jj = pl.ds(j * LANES, LANES)
                gidx[jj] = sdb[jj] + c * N
                return 0
            lax.fori_loop(0, NSB, _idx, 0)

        def _mk_didx(sdb):
            def _idx(j, _):
                jj = pl.ds(j * LANES, LANES)
                didx[jj] = sdb[pl.ds(G + j * LANES, LANES)]
                return 0
            lax.fori_loop(0, NSB, _idx, 0)

        def _gather_wait(gidx, rows, sem):
            pltpu.make_async_copy(h_hbm.at[gidx], rows, sem).wait()

        def _scale(g, rows):
            def _sc16(sb, _):
                av16 = exv[pl.ds(g * G + sb * LANES, LANES)]
                for r2 in range(LANES):
                    a = jnp.full((LANES,), av16[r2], jnp.float32)
                    row = sb * LANES + r2
                    for j in range(8):
                        jj = pl.ds(j * LANES, LANES)
                        rows[row, jj] = rows[row, jj] * a
                return 0
            lax.fori_loop(0, NSB, _sc16, 0)

        for cl in range(cph):
            c = core * cph + cl
            pltpu.sync_copy(z_hbm, acc_sh.at[pl.ds(s * STRIPE, STRIPE)])
            plsc.subcore_barrier()

            _sd_start(0, sdb0, sem_s0)
            _sd_start(1, sdb1, sem_s1)
            _sd_wait(sdb0, sem_s0)
            _mk_gidx(sdb0, gidx0, c)
            pltpu.async_copy(h_hbm.at[gidx0], rows0, sem_g0)

            def _pair2(k, _):
                g = 2 * k
                _sd_wait(sdb1, sem_s1)
                _mk_gidx(sdb1, gidx1, c)
                pltpu.async_copy(h_hbm.at[gidx1], rows1, sem_g1)
                _mk_didx(sdb0)

                @pl.when(k < npairs - 1)
                def _():
                    _sd_start(g + 2, sdb0, sem_s0)
                _gather_wait(gidx0, rows0, sem_g0)
                _scale(g, rows0)
                pltpu.sync_copy(rows0, acc_sh.at[didx], add=True)

                @pl.when(k < npairs - 1)
                def _():
                    _sd_wait(sdb0, sem_s0)
                    _mk_gidx(sdb0, gidx0, c)
                    pltpu.async_copy(h_hbm.at[gidx0], rows0, sem_g0)
                _mk_didx(sdb1)

                @pl.when(k < npairs - 1)
                def _():
                    _sd_start(g + 3, sdb1, sem_s1)
                _gather_wait(gidx1, rows1, sem_g1)
                _scale(g + 1, rows1)
                pltpu.sync_copy(rows1, acc_sh.at[didx], add=True)
                return 0
            lax.fori_loop(0, npairs, _pair2, 0)
            plsc.subcore_barrier()
            pltpu.sync_copy(acc_sh.at[pl.ds(s * STRIPE, STRIPE)],
                            out_hbm.at[c, pl.ds(s * STRIPE, STRIPE)])
            plsc.subcore_barrier()

    pl.run_scoped(_phase2,
                  pltpu.VMEM((G, 128), jnp.float32),
                  pltpu.VMEM((G, 128), jnp.float32))


def _gat_sc(hflat, as_f, ad_f, sd, zrows, ch, cph, ec, e_tot):
    mesh = plsc.VectorSubcoreMesh(
        core_axis_name="c", subcore_axis_name="s", num_cores=NC, num_subcores=NS)
    body = functools.partial(_gat_sc_body, ch, cph, ec, e_tot)
    return pl.kernel(
        body,
        out_type=jax.ShapeDtypeStruct((ch, NP, 128), jnp.float32),
        mesh=mesh,
        compiler_params=pltpu.CompilerParams(needs_layout_passes=False),
        scratch_types=[
            pltpu.VMEM((NP // 128, 128), jnp.float32),
            pltpu.VMEM((ec,), jnp.float32),
            pltpu.VMEM((2 * G,), jnp.int32),
            pltpu.VMEM((2 * G,), jnp.int32),
            pltpu.VMEM((G,), jnp.int32),
            pltpu.VMEM((G,), jnp.int32),
            pltpu.VMEM((G,), jnp.int32),
            pltpu.VMEM((LANES,), jnp.int32),
            pltpu.VMEM((LANES,), jnp.float32),
            pltpu.VMEM_SHARED((NP, 128), jnp.float32),
            pltpu.SemaphoreType.DMA,
            pltpu.SemaphoreType.DMA,
            pltpu.SemaphoreType.DMA,
            pltpu.SemaphoreType.DMA,
        ],
    )(hflat, as_f, ad_f, sd, zrows)


def kernel(x, edge_index, W1, as1, ad1, b1, W2, as2, ad2, b2,
           Wm, asm_, adm, bm, Wl, asl, adl, bl):
    n = x.shape[0]
    e_in = edge_index.shape[1]
    e_tot = e_in + n
    ec = -(-e_tot // (NS * 2 * G)) * 2 * G
    e_pad = ec * NS

    loops = jnp.arange(n, dtype=jnp.int32)
    pad = jnp.zeros((e_pad - e_tot,), jnp.int32)
    src = jnp.concatenate([edge_index[0].astype(jnp.int32), loops, pad])
    dst = jnp.concatenate([edge_index[1].astype(jnp.int32), loops, pad])
    sd = jnp.concatenate(
        [src.reshape(-1, G), dst.reshape(-1, G)], axis=1).reshape(-1)
    zrows = jnp.zeros((STRIPE, 128), jnp.float32)

    def gat_layer(xin, w, a_s, a_d, ch, cph):
        h, vs, vd = _matmul_att(xin, w, a_s.reshape(ch, 128), a_d.reshape(ch, 128),
                                ch, cph)
        pad_n = ((0, 0), (0, NP - n))
        vs_f = jnp.pad(vs.T, pad_n).reshape(-1)
        vd_f = jnp.pad(vd.T, pad_n).reshape(-1)
        agg = _gat_sc(h.reshape(ch * n, 128), vs_f, vd_f, sd, zrows,
                      ch, cph, ec, e_tot)
        return agg[:, :n, :]

    agg1 = gat_layer(x, W1, as1, ad1, ch=8, cph=4)
    x2 = _finish(agg1, b1.reshape(8, 128), do_relu=True)
    agg2 = gat_layer(x2, W2, as2, ad2, ch=8, cph=4)
    x3 = _finish(agg2, b2.reshape(8, 128), do_relu=True)

    wml = jnp.concatenate([Wm, Wl], axis=1)
    asml = jnp.concatenate([asm_.reshape(1, 256), asl.reshape(1, 256)], axis=0)
    adml = jnp.concatenate([adm.reshape(1, 256), adl.reshape(1, 256)], axis=0)
    aggml = gat_layer(x3, wml, asml, adml, ch=4, cph=2)
    mu = _finish(aggml[0:2], bm.reshape(2, 128), do_relu=False)
    logstd = _finish(aggml[2:4], bl.reshape(2, 128), do_relu=False)
    return (mu, logstd)


# speedup vs baseline: 19.0388x; 1.0117x over previous
import functools

import jax
import jax.numpy as jnp
from jax import lax
from jax.experimental import pallas as pl
from jax.experimental.pallas import tpu as pltpu
from jax.experimental.pallas import tpu_sc as plsc

N = 10000
NP = 10240
NC = 2
NS = 16
LANES = 16
G = 96
BLK = 1000
STRIPE = NP // NS


def _mm_body(cph, x_ref, w_ref, as_ref, ad_ref, h_ref, s_ref, d_ref):
    c = pl.program_id(1)
    hv = jnp.dot(x_ref[...].astype(jnp.bfloat16), w_ref[...].astype(jnp.bfloat16),
                 preferred_element_type=jnp.float32)
    h_ref[0] = hv
    head = c // cph
    nh = s_ref.shape[1]
    onehot = (lax.broadcasted_iota(jnp.int32, (1, nh), 1) == head).astype(jnp.float32)
    sc = jnp.sum(hv * as_ref[pl.ds(c, 1), :], axis=1, keepdims=True)
    dc = jnp.sum(hv * ad_ref[pl.ds(c, 1), :], axis=1, keepdims=True)

    @pl.when(c == 0)
    def _():
        s_ref[...] = jnp.zeros_like(s_ref)
        d_ref[...] = jnp.zeros_like(d_ref)

    s_ref[...] += sc * onehot
    d_ref[...] += dc * onehot


def _matmul_att(x, w, asv, adv, ch, cph):
    n, k = x.shape
    nh = ch // cph
    nb = n // BLK
    return pl.pallas_call(
        functools.partial(_mm_body, cph),
        grid=(nb, ch),
        in_specs=[
            pl.BlockSpec((BLK, k), lambda i, c: (i, 0)),
            pl.BlockSpec((k, 128), lambda i, c: (0, c)),
            pl.BlockSpec((ch, 128), lambda i, c: (0, 0)),
            pl.BlockSpec((ch, 128), lambda i, c: (0, 0)),
        ],
        out_specs=[
            pl.BlockSpec((1, BLK, 128), lambda i, c: (c, i, 0)),
            pl.BlockSpec((BLK, nh), lambda i, c: (i, 0)),
            pl.BlockSpec((BLK, nh), lambda i, c: (i, 0)),
        ],
        out_shape=[
            jax.ShapeDtypeStruct((ch, n, 128), jnp.float32),
            jax.ShapeDtypeStruct((n, nh), jnp.float32),
            jax.ShapeDtypeStruct((n, nh), jnp.float32),
        ],
    )(x, w, asv, adv)


def _fin_body(do_relu, a_ref, b_ref, o_ref):
    c = pl.program_id(1)
    v = a_ref[0] + b_ref[pl.ds(c, 1), :]
    if do_relu:
        v = jnp.maximum(v, 0.0)
    o_ref[...] = v


def _finish(agg, b2d, do_relu, ch, coff=0):
    nb = N // BLK
    return pl.pallas_call(
        functools.partial(_fin_body, do_relu),
        grid=(nb, ch),
        in_specs=[
            pl.BlockSpec((1, BLK, 128), lambda i, c: (c + coff, i, 0)),
            pl.BlockSpec((ch, 128), lambda i, c: (0, 0)),
        ],
        out_specs=pl.BlockSpec((BLK, 128), lambda i, c: (i, c)),
        out_shape=jax.ShapeDtypeStruct((N, ch * 128), jnp.float32),
    )(agg, b2d)


def _gat_sc_body(ch, cph, ec, e_tot,
                 h_hbm, as_hbm, ad_hbm, sd_hbm, z_hbm, out_hbm,
                 den2, exv, sdb0, sdb1, gidx0, gidx1, didx, kbuf, cbuf,
                 acc_sh, sem_s0, sem_s1, sem_g0, sem_g1):
    core = lax.axis_index("c")
    s = lax.axis_index("s")
    head = core
    ngroups = ec // G
    npairs = ngroups // 2
    ebase = s * ec
    sdbase = s * ngroups * 2 * G
    io16 = lax.broadcasted_iota(jnp.int32, (LANES,), 0)
    NSB = G // LANES

    def _sd_start(g, sdb, sem):
        pltpu.async_copy(sd_hbm.at[pl.ds(sdbase + g * 2 * G, 2 * G)], sdb, sem)

    def _sd_wait(sdb, sem):
        pltpu.make_async_copy(sd_hbm.at[pl.ds(0, 2 * G)], sdb, sem).wait()

    def _allmax(v):
        for k in (1, 2, 4, 8):
            cbuf[...] = v
            v = jnp.maximum(v, plsc.load_gather(cbuf, [io16 ^ k]))
        return v

    def _cumsum16(v):
        for k in (1, 2, 4, 8):
            cbuf[...] = v
            t = plsc.load_gather(cbuf, [jnp.maximum(io16 - k, 0)])
            v = v + jnp.where(io16 >= k, t, 0.0)
        return v

    def _cummax16i(v):
        for k in (1, 2, 4, 8):
            kbuf[...] = v
            t = plsc.load_gather(kbuf, [jnp.maximum(io16 - k, 0)])
            v = jnp.maximum(v, jnp.where(io16 >= k, t, 0))
        return v

    def _z_den(i, _):
        den2[i // 8, pl.ds((i % 8) * LANES, LANES)] = jnp.zeros(
            (LANES,), jnp.float32)
        return 0
    lax.fori_loop(0, NP // LANES, _z_den, 0)

    def _phase1(asrc_t, adst_t):
        pltpu.sync_copy(as_hbm.at[pl.ds(head * NP, NP)], asrc_t)
        pltpu.sync_copy(ad_hbm.at[pl.ds(head * NP, NP)], adst_t)

        def _mx(i, carry):
            ms, md = carry
            ms = jnp.maximum(ms, asrc_t[pl.ds(i * LANES, LANES)])
            md = jnp.maximum(md, adst_t[pl.ds(i * LANES, LANES)])
            return ms, md
        neg = jnp.full((LANES,), -3.0e38, jnp.float32)
        ms, md = lax.fori_loop(0, N // LANES, _mx, (neg, neg))
        m_bound = jnp.maximum(_allmax(ms)[0] + _allmax(md)[0], 0.0)

        def _proc1(g, sdb):
            def _inner(sb, _):
                ii = pl.ds(g * G + sb * LANES, LANES)
                sv = sdb[pl.ds(sb * LANES, LANES)]
                dv = sdb[pl.ds(G + sb * LANES, LANES)]
                av = plsc.load_gather(asrc_t, [sv])
                bv = plsc.load_gather(adst_t, [dv])
                e = av + bv
                e = jnp.where(e >= 0.0, e, 0.2 * e)
                exu = jnp.exp(e - m_bound)
                eid = ebase + g * G + sb * LANES + io16
                exu = jnp.where(eid < e_tot, exu, 0.0)
                exv[ii] = exu
                kk, vv = plsc.sort_key_val(dv, exu)
                cs = _cumsum16(vv)
                kbuf[...] = kk
                knext = plsc.load_gather(
                    kbuf, [jnp.minimum(io16 + 1, LANES - 1)])
                kprev = plsc.load_gather(kbuf, [jnp.maximum(io16 - 1, 0)])
                is_last = (kk != knext) | (io16 == LANES - 1)
                is_first = (kk != kprev) | (io16 == 0)
                start = _cummax16i(jnp.where(is_first, io16, 0))
                cbuf[...] = cs
                base = plsc.load_gather(cbuf, [jnp.maximum(start - 1, 0)])
                base = jnp.where(start > 0, base, 0.0)
                plsc.addupdate_scatter(
                    den2, [lax.shift_right_logical(kk, 7), kk & 127],
                    cs - base, mask=is_last)
                return 0
            lax.fori_loop(0, NSB, _inner, 0)

        _sd_start(0, sdb0, sem_s0)
        _sd_start(1, sdb1, sem_s1)

        def _pair(k, _):
            g = 2 * k
            _sd_wait(sdb0, sem_s0)
            _proc1(g, sdb0)

            @pl.when(k < npairs - 1)
            def _():
                _sd_start(g + 2, sdb0, sem_s0)
            _sd_wait(sdb1, sem_s1)
            _proc1(g + 1, sdb1)

            @pl.when(k < npairs - 1)
            def _():
                _sd_start(g + 3, sdb1, sem_s1)
            return 0
        lax.fori_loop(0, npairs, _pair, 0)

    pl.run_scoped(_phase1,
                  pltpu.VMEM((NP,), jnp.float32),
                  pltpu.VMEM((NP,), jnp.float32))

    def _phase2(rows0, rows1):
        DB = NP // 128
        DR = STRIPE // 128
        pltpu.sync_copy(den2, acc_sh.at[pl.ds(s * DB, DB)])
        plsc.subcore_barrier()
        pltpu.sync_copy(acc_sh.at[pl.ds(s * DR, DR)], rows0.at[pl.ds(0, DR)])
        for t in range(1, NS):
            pltpu.sync_copy(acc_sh.at[pl.ds(t * DB + s * DR, DR)],
                            rows0.at[pl.ds(8, DR)])

            def _acc(i, _):
                r, j = i // 8, i % 8
                jj = pl.ds(j * LANES, LANES)
                rows0[r, jj] += rows0[8 + r, jj]
                return 0
            lax.fori_loop(0, DR * 8, _acc, 0)
        plsc.subcore_barrier()
        pltpu.sync_copy(rows0.at[pl.ds(0, DR)], acc_sh.at[pl.ds(s * DR, DR)])
        plsc.subcore_barrier()
        pltpu.sync_copy(acc_sh.at[pl.ds(0, DB)], den2)

        def _procc(g, sdb):
            def _inner(sb, _):
                ii = pl.ds(g * G + sb * LANES, LANES)
                dv = sdb[pl.ds(G + sb * LANES, LANES)]
                den_v = plsc.load_gather(
                    den2, [lax.shift_right_logical(dv, 7), dv & 127])
                exv[ii] = exv[ii] / (den_v + 1e-16)
                return 0
            lax.fori_loop(0, NSB, _inner, 0)

        _sd_start(0, sdb0, sem_s0)
        _sd_start(1, sdb1, sem_s1)

        def _pairc(k, _):
            g = 2 * k
            _sd_wait(sdb0, sem_s0)
            _procc(g, sdb0)

            @pl.when(k < npairs - 1)
            def _():
                _sd_start(g + 2, sdb0, sem_s0)
            _sd_wait(sdb1, sem_s1)
            _procc(g + 1, sdb1)

            @pl.when(k < npairs - 1)
            def _():
                _sd_start(g + 3, sdb1, sem_s1)
            return 0
        lax.fori_loop(0, npairs, _pairc, 0)

        def _mk_gidx(sdb, gidx, c):
            def _idx(j, _):
                jj = pl.ds(j * LANES, LANES)
                gidx[jj] = sdb[jj] + c * N
                return 0
            lax.fori_loop(0, NSB, _idx, 0)

        def _mk_didx(sdb):
            def _idx(j, _):
                jj = pl.ds(j * LANES, LANES)
                didx[jj] = sdb[pl.ds(G + j * LANES, LANES)]
                return 0
            lax.fori_loop(0, NSB, _idx, 0)

        def _gather_wait(gidx, rows, sem):
            pltpu.make_async_copy(h_hbm.at[gidx], rows, sem).wait()

        def _scale(g, rows):
            def _sc16(sb, _):
                av16 = exv[pl.ds(g * G + sb * LANES, LANES)]
                for r2 in range(LANES):
                    a = jnp.full((LANES,), av16[r2], jnp.float32)
                    row = sb * LANES + r2
                    for j in range(8):
                        jj = pl.ds(j * LANES, LANES)
                        rows[row, jj] = rows[row, jj] * a
                return 0
            lax.fori_loop(0, NSB, _sc16, 0)

        for cl in range(cph):
            c = core * cph + cl
            pltpu.sync_copy(z_hbm, acc_sh.at[pl.ds(s * STRIPE, STRIPE)])
            plsc.subcore_barrier()

            _sd_start(0, sdb0, sem_s0)
            _sd_start(1, sdb1, sem_s1)
            _sd_wait(sdb0, sem_s0)
            _mk_gidx(sdb0, gidx0, c)
            pltpu.async_copy(h_hbm.at[gidx0], rows0, sem_g0)

            def _pair2(k, _):
                g = 2 * k
                _sd_wait(sdb1, sem_s1)
                _mk_gidx(sdb1, gidx1, c)
                pltpu.async_copy(h_hbm.at[gidx1], rows1, sem_g1)
                _mk_didx(sdb0)

                @pl.when(k < npairs - 1)
                def _():
                    _sd_start(g + 2, sdb0, sem_s0)
                _gather_wait(gidx0, rows0, sem_g0)
                _scale(g, rows0)
                pltpu.sync_copy(rows0, acc_sh.at[didx], add=True)

                @pl.when(k < npairs - 1)
                def _():
                    _sd_wait(sdb0, sem_s0)
                    _mk_gidx(sdb0, gidx0, c)
                    pltpu.async_copy(h_hbm.at[gidx0], rows0, sem_g0)
                _mk_didx(sdb1)

                @pl.when(k < npairs - 1)
                def _():
                    _sd_start(g + 3, sdb1, sem_s1)
                _gather_wait(gidx1, rows1, sem_g1)
                _scale(g + 1, rows1)
                pltpu.sync_copy(rows1, acc_sh.at[didx], add=True)
                return 0
            lax.fori_loop(0, npairs, _pair2, 0)
            plsc.subcore_barrier()
            pltpu.sync_copy(acc_sh.at[pl.ds(s * STRIPE, STRIPE)],
                            out_hbm.at[c, pl.ds(s * STRIPE, STRIPE)])
            plsc.subcore_barrier()

    pl.run_scoped(_phase2,
                  pltpu.VMEM((G, 128), jnp.float32),
                  pltpu.VMEM((G, 128), jnp.float32))


def _gat_sc(hflat, as_f, ad_f, sd, zrows, ch, cph, ec, e_tot):
    mesh = plsc.VectorSubcoreMesh(
        core_axis_name="c", subcore_axis_name="s", num_cores=NC, num_subcores=NS)
    body = functools.partial(_gat_sc_body, ch, cph, ec, e_tot)
    return pl.kernel(
        body,
        out_type=jax.ShapeDtypeStruct((ch, NP, 128), jnp.float32),
        mesh=mesh,
        compiler_params=pltpu.CompilerParams(needs_layout_passes=False),
        scratch_types=[
            pltpu.VMEM((NP // 128, 128), jnp.float32),
            pltpu.VMEM((ec,), jnp.float32),
            pltpu.VMEM((2 * G,), jnp.int32),
            pltpu.VMEM((2 * G,), jnp.int32),
            pltpu.VMEM((G,), jnp.int32),
            pltpu.VMEM((G,), jnp.int32),
            pltpu.VMEM((G,), jnp.int32),
            pltpu.VMEM((LANES,), jnp.int32),
            pltpu.VMEM((LANES,), jnp.float32),
            pltpu.VMEM_SHARED((NP, 128), jnp.float32),
            pltpu.SemaphoreType.DMA,
            pltpu.SemaphoreType.DMA,
            pltpu.SemaphoreType.DMA,
            pltpu.SemaphoreType.DMA,
        ],
    )(hflat, as_f, ad_f, sd, zrows)


def kernel(x, edge_index, W1, as1, ad1, b1, W2, as2, ad2, b2,
           Wm, asm_, adm, bm, Wl, asl, adl, bl):
    n = x.shape[0]
    e_in = edge_index.shape[1]
    e_tot = e_in + n
    ec = -(-e_tot // (NS * 2 * G)) * 2 * G
    e_pad = ec * NS

    loops = jnp.arange(n, dtype=jnp.int32)
    pad = jnp.zeros((e_pad - e_tot,), jnp.int32)
    src = jnp.concatenate([edge_index[0].astype(jnp.int32), loops, pad])
    dst = jnp.concatenate([edge_index[1].astype(jnp.int32), loops, pad])
    sd = jnp.concatenate(
        [src.reshape(-1, G), dst.reshape(-1, G)], axis=1).reshape(-1)
    zrows = jnp.zeros((STRIPE, 128), jnp.float32)

    def gat_layer(xin, w, a_s, a_d, ch, cph):
        h, vs, vd = _matmul_att(xin, w, a_s.reshape(ch, 128), a_d.reshape(ch, 128),
                                ch, cph)
        pad_n = ((0, 0), (0, NP - n))
        vs_f = jnp.pad(vs.T, pad_n).reshape(-1)
        vd_f = jnp.pad(vd.T, pad_n).reshape(-1)
        return _gat_sc(h.reshape(ch * n, 128), vs_f, vd_f, sd, zrows,
                       ch, cph, ec, e_tot)

    agg1 = gat_layer(x, W1, as1, ad1, ch=8, cph=4)
    x2 = _finish(agg1, b1.reshape(8, 128), do_relu=True, ch=8)
    agg2 = gat_layer(x2, W2, as2, ad2, ch=8, cph=4)
    x3 = _finish(agg2, b2.reshape(8, 128), do_relu=True, ch=8)

    wml = jnp.concatenate([Wm, Wl], axis=1)
    asml = jnp.concatenate([asm_.reshape(1, 256), asl.reshape(1, 256)], axis=0)
    adml = jnp.concatenate([adm.reshape(1, 256), adl.reshape(1, 256)], axis=0)
    aggml = gat_layer(x3, wml, asml, adml, ch=4, cph=2)
    mu = _finish(aggml, bm.reshape(2, 128), do_relu=False, ch=2, coff=0)
    logstd = _finish(aggml, bl.reshape(2, 128), do_relu=False, ch=2, coff=2)
    return (mu, logstd)
